# baseline (speedup 1.0000x reference)
import functools
import jax
import jax.numpy as jnp
from jax import lax
from jax.experimental import pallas as pl
from jax.experimental.pallas import tpu as pltpu
from jax.experimental.pallas import tpu_sc as plsc

_T, _N, _D, _M = 2, 100, 128, 1024
_R = _T * _N
_MARGIN = 0.5

_NC, _NS, _L = 2, 16, 16
_NW = _NC * _NS
_ROWS_PER_W = 8
_RP = _NW * _ROWS_PER_W


def _l2n(x):
    return x / jnp.maximum(jnp.sqrt(jnp.sum(x * x, axis=-1, keepdims=True)), 1e-12)


def _tc_body(emb_ref, glo_ref, th_ref, k_ref, v_ref,
             nemb_ref, eg_ref, skv_ref, svk_ref, lk_ref, lv_ref):
    ne = _l2n(emb_ref[...])
    ng = _l2n(glo_ref[...])
    nemb_ref[...] = ne

    kmat = k_ref[...]
    vmat = v_ref[...]
    sim_kv = jax.lax.dot_general(ne, kmat, (((1,), (1,)), ((), ())),
                                 preferred_element_type=jnp.float32)
    sim_vk = jax.lax.dot_general(ng, vmat, (((1,), (1,)), ((), ())),
                                 preferred_element_type=jnp.float32)
    skv_ref[0:_R, :] = sim_kv
    skv_ref[_R:_RP, :] = jnp.zeros((_RP - _R, _M), jnp.float32)
    svk_ref[0:_R, :] = sim_vk
    svk_ref[_R:_RP, :] = jnp.zeros((_RP - _R, _M), jnp.float32)

    th0 = th_ref[0]
    th1 = th_ref[1]
    th2 = th_ref[2]
    th3 = th_ref[3]

    pos_score = jnp.where(sim_kv > th0, sim_kv, 0.0)
    eg = ng + jax.lax.dot_general(pos_score, vmat, (((1,), (0,)), ((), ())),
                                  preferred_element_type=jnp.float32)
    eg_ref[...] = _l2n(eg)

    iota = jax.lax.broadcasted_iota(jnp.int32, (_R, _M), 1)
    big = jnp.int32(2 ** 30)
    inf = jnp.float32(jnp.inf)

    def pair_contrib(src, other, thp, thn):
        mp = jnp.where(src > thp, src, inf)
        mn = jnp.where(src < thn, src, -inf)
        extp = jnp.min(mp, axis=1, keepdims=True)
        extn = jnp.max(mn, axis=1, keepdims=True)
        idxp = jnp.min(jnp.where(mp == extp, iota, big), axis=1, keepdims=True)
        idxn = jnp.min(jnp.where(mn == extn, iota, big), axis=1, keepdims=True)
        valp = jnp.sum(jnp.where(iota == idxp, other, 0.0), axis=1, keepdims=True)
        valn = jnp.sum(jnp.where(iota == idxn, other, 0.0), axis=1, keepdims=True)
        anyp = (extp != inf).astype(jnp.float32)
        anyn = (extn != -inf).astype(jnp.float32)
        return jnp.sum(anyp * valp - anyn * valn)

    lv_ref[0] = jnp.maximum(
        -pair_contrib(sim_kv, sim_vk, th0, th1) / _R + _MARGIN, 0.0)
    lk_ref[0] = jnp.maximum(
        -pair_contrib(sim_vk, sim_kv, th2, th3) / _R + _MARGIN, 0.0)


def _sc_loss_s(skv, svk):
    mesh = plsc.VectorSubcoreMesh(core_axis_name="c", subcore_axis_name="s")

    @functools.partial(
        pl.kernel, mesh=mesh,
        out_type=jax.ShapeDtypeStruct((_NW, _L), jnp.float32),
        scratch_types=[
            pltpu.VMEM((_ROWS_PER_W, _M), jnp.float32),
            pltpu.VMEM((_ROWS_PER_W, _M), jnp.float32),
            pltpu.VMEM((_L,), jnp.float32),
        ],
    )
    def k(skv_hbm, svk_hbm, out_hbm, a_v, b_v, acc_v):
        wid = lax.axis_index("s") * _NC + lax.axis_index("c")
        base = wid * _ROWS_PER_W
        pltpu.sync_copy(skv_hbm.at[pl.ds(base, _ROWS_PER_W)], a_v)
        pltpu.sync_copy(svk_hbm.at[pl.ds(base, _ROWS_PER_W)], b_v)

        def row_body(r, acc):
            def chunk_body(i, a2):
                a = a_v[r, pl.ds(i * _L, _L)]
                b = b_v[r, pl.ds(i * _L, _L)]
                d = b - a
                return a2 + d * d
            return lax.fori_loop(0, _M // _L, chunk_body, acc)

        acc_v[...] = lax.fori_loop(0, _ROWS_PER_W, row_body,
                                   jnp.zeros((_L,), jnp.float32))
        pltpu.sync_copy(acc_v, out_hbm.at[wid])

    partials = k(skv, svk)
    return jnp.sum(partials) / (_R * _M)


def kernel(emb_support, emb_query, glo_support, glo_query, thresh,
           memory_keys, memory_values):
    emb = jnp.concatenate([emb_support, emb_query], axis=1).reshape(_R, _D)
    glo = jnp.concatenate([glo_support, glo_query], axis=1).reshape(_R, _D)

    out_shape = (
        jax.ShapeDtypeStruct((_R, _D), jnp.float32),
        jax.ShapeDtypeStruct((_R, _D), jnp.float32),
        jax.ShapeDtypeStruct((_RP, _M), jnp.float32),
        jax.ShapeDtypeStruct((_RP, _M), jnp.float32),
        jax.ShapeDtypeStruct((1,), jnp.float32),
        jax.ShapeDtypeStruct((1,), jnp.float32),
    )
    vspec = pl.BlockSpec(memory_space=pltpu.VMEM)
    sspec = pl.BlockSpec(memory_space=pltpu.SMEM)
    in_specs = [vspec, vspec, sspec, vspec, vspec]
    out_specs = (vspec, vspec, vspec, vspec, sspec, sspec)
    ne, eg, skv, svk, lk, lv = pl.pallas_call(
        _tc_body,
        out_shape=out_shape,
        in_specs=in_specs,
        out_specs=out_specs,
    )(emb, glo, thresh, memory_keys, memory_values)

    ls = _sc_loss_s(skv, svk)

    return (ne.reshape(_T, _N, _D), eg.reshape(_T, _N, _D),
            lk[0], lv[0], ls)


# speedup vs baseline: 3.2367x; 3.2367x over previous
import jax
import jax.numpy as jnp
from jax.experimental import pallas as pl
from jax.experimental.pallas import tpu as pltpu

_T, _N, _D, _M = 2, 100, 128, 1024
_R = _T * _N
_MARGIN = 0.5


def _l2n(x):
    return x / jnp.maximum(jnp.sqrt(jnp.sum(x * x, axis=-1, keepdims=True)), 1e-12)


def _body(emb_ref, glo_ref, th_ref, k_ref, v_ref,
          nemb_ref, eg_ref, lk_ref, lv_ref, ls_ref):
    ne = _l2n(emb_ref[...])
    ng = _l2n(glo_ref[...])
    nemb_ref[...] = ne

    kmat = k_ref[...]
    vmat = v_ref[...]
    sim_kv = jax.lax.dot_general(ne, kmat, (((1,), (1,)), ((), ())),
                                 preferred_element_type=jnp.float32)
    sim_vk = jax.lax.dot_general(ng, vmat, (((1,), (1,)), ((), ())),
                                 preferred_element_type=jnp.float32)

    th0 = th_ref[0]
    th1 = th_ref[1]
    th2 = th_ref[2]
    th3 = th_ref[3]

    pos_score = jnp.where(sim_kv > th0, sim_kv, 0.0)
    eg = ng + jax.lax.dot_general(pos_score, vmat, (((1,), (0,)), ((), ())),
                                  preferred_element_type=jnp.float32)
    eg_ref[...] = _l2n(eg)

    diff = sim_vk - sim_kv
    ls_ref[0] = jnp.sum(diff * diff) / (_R * _M)

    iota = jax.lax.broadcasted_iota(jnp.int32, (_R, _M), 1)
    big = jnp.int32(2 ** 30)
    inf = jnp.float32(jnp.inf)

    def pair_contrib(src, other, thp, thn):
        mp = jnp.where(src > thp, src, inf)
        mn = jnp.where(src < thn, src, -inf)
        extp = jnp.min(mp, axis=1, keepdims=True)
        extn = jnp.max(mn, axis=1, keepdims=True)
        idxp = jnp.min(jnp.where(mp == extp, iota, big), axis=1, keepdims=True)
        idxn = jnp.min(jnp.where(mn == extn, iota, big), axis=1, keepdims=True)
        valp = jnp.sum(jnp.where(iota == idxp, other, 0.0), axis=1, keepdims=True)
        valn = jnp.sum(jnp.where(iota == idxn, other, 0.0), axis=1, keepdims=True)
        anyp = (extp != inf).astype(jnp.float32)
        anyn = (extn != -inf).astype(jnp.float32)
        return jnp.sum(anyp * valp - anyn * valn)

    lv_ref[0] = jnp.maximum(
        -pair_contrib(sim_kv, sim_vk, th0, th1) / _R + _MARGIN, 0.0)
    lk_ref[0] = jnp.maximum(
        -pair_contrib(sim_vk, sim_kv, th2, th3) / _R + _MARGIN, 0.0)


def kernel(emb_support, emb_query, glo_support, glo_query, thresh,
           memory_keys, memory_values):
    emb = jnp.concatenate([emb_support, emb_query], axis=1).reshape(_R, _D)
    glo = jnp.concatenate([glo_support, glo_query], axis=1).reshape(_R, _D)

    out_shape = (
        jax.ShapeDtypeStruct((_R, _D), jnp.float32),
        jax.ShapeDtypeStruct((_R, _D), jnp.float32),
        jax.ShapeDtypeStruct((1,), jnp.float32),
        jax.ShapeDtypeStruct((1,), jnp.float32),
        jax.ShapeDtypeStruct((1,), jnp.float32),
    )
    vspec = pl.BlockSpec(memory_space=pltpu.VMEM)
    sspec = pl.BlockSpec(memory_space=pltpu.SMEM)
    in_specs = [vspec, vspec, sspec, vspec, vspec]
    out_specs = (vspec, vspec, sspec, sspec, sspec)
    ne, eg, lk, lv, ls = pl.pallas_call(
        _body,
        out_shape=out_shape,
        in_specs=in_specs,
        out_specs=out_specs,
    )(emb, glo, thresh, memory_keys, memory_values)

    return (ne.reshape(_T, _N, _D), eg.reshape(_T, _N, _D),
            lk[0], lv[0], ls[0])
